# baseline (speedup 1.0000x reference)
import functools

import jax
import jax.numpy as jnp
from jax import lax
from jax.experimental import pallas as pl
from jax.experimental.pallas import tpu as pltpu
from jax.experimental.pallas import tpu_sc as plsc

MASK_RATIO = 0.5
_NC = 2
_NS = 16
_LN = 16
_CTLW = 64


def _energy_body(x_ref, e_ref, *, inv_c):
    e = jnp.sum(x_ref[0], axis=0) * inv_c
    bu = lax.bitcast_convert_type(e, jnp.uint32)
    key = jnp.where(bu >= jnp.uint32(0x80000000), ~bu,
                    bu | jnp.uint32(0x80000000))
    e_ref[...] = key[None]


def _sc_select_body(keys_hbm, mask_hbm, data_v, mask_v, hist_v, hist2_v,
                    lead_v, hsum_v, ctl_v, eqc_loc_v, eqc_all_v, nl_v, row_v,
                    sh_hist, sh_ctl, sh_eqc, *, bpc, rps, w, k):
    w16 = w // _LN
    nbin = 256
    sid = lax.axis_index("s")
    core = lax.axis_index("c")
    b0 = core * bpc
    r0 = sid * rps
    lane = lax.iota(jnp.int32, _LN)

    pltpu.sync_copy(keys_hbm.at[pl.ds(b0, bpc), pl.ds(r0, rps), :], data_v)

    for t in range(4):
        sh = 24 - 8 * t
        for j in range((bpc * nbin) // _LN):
            hist_v[pl.ds(j * _LN, _LN)] = jnp.zeros((_LN,), jnp.int32)

        def scan_bb(bb, _, t=t, sh=sh):
            if t > 0:
                pref_hi = ctl_v[bb, pl.ds(0, _LN)][0] >> jnp.uint32(sh + 8)

            def scan_r(r, _):
                for g in range(w16):
                    kv = data_v[bb, r, pl.ds(g * _LN, _LN)]
                    digits = ((kv >> jnp.uint32(sh)) & jnp.uint32(nbin - 1)
                              ).astype(jnp.int32) + bb * nbin
                    if t > 0:
                        match = (kv >> jnp.uint32(sh + 8)) == pref_hi
                        cnts, lastm = plsc.scan_count(digits, match)
                    else:
                        cnts, lastm = plsc.scan_count(digits)
                    plsc.addupdate_scatter(hist_v, [digits], cnts, mask=lastm)
                return 0

            lax.fori_loop(0, rps, scan_r, 0, unroll=2)
            return 0

        lax.fori_loop(0, bpc, scan_bb, 0)

        for j in range((bpc * nbin) // _LN):
            sl = pl.ds(j * _LN, _LN)
            hist2_v[sl] = hist_v[sl]
        pltpu.sync_copy(hist2_v, sh_hist.at[sid])
        plsc.subcore_barrier()

        @pl.when(sid < bpc)
        def _leader(t=t, sh=sh):
            bb = sid
            pltpu.sync_copy(sh_hist.at[:, pl.ds(bb * nbin, nbin)], lead_v)
            for j in range(nbin // _LN):
                hsum_v[pl.ds(j * _LN, _LN)] = jnp.zeros((_LN,), jnp.int32)

            def accum(rr, _):
                for j in range(nbin // _LN):
                    sl = pl.ds(j * _LN, _LN)
                    hsum_v[sl] = hsum_v[sl] + lead_v[rr, sl]
                return 0

            lax.fori_loop(0, _NS, accum, 0)

            if t == 0:
                pref = jnp.uint32(0)
                krem = jnp.int32(k)
            else:
                rowc = ctl_v[bb, pl.ds(0, _LN)]
                pref = rowc[0]
                krem = rowc[1].astype(jnp.int32)

            def suffix(j, carry):
                acc, found, binsel, above = carry
                start = (nbin // _LN - 1 - j) * _LN
                v = hsum_v[pl.ds(start, _LN)]
                rv = lax.rev(v, (0,))
                s = plsc.cumsum(rv) + acc
                mlt = s < krem
                nlt = jnp.sum(mlt.astype(jnp.int32))
                tot = jnp.sum(v)
                hit = ((acc + tot) >= krem) & jnp.logical_not(found)
                bin_new = start + (_LN - 1) - nlt
                above_new = acc + jnp.sum(jnp.where(mlt, rv, 0))
                return (acc + tot, found | hit,
                        jnp.where(hit, bin_new, binsel),
                        jnp.where(hit, above_new, above))

            _, _, binsel, above = lax.fori_loop(
                0, nbin // _LN, suffix,
                (jnp.int32(0), False, jnp.int32(0), jnp.int32(0)))
            prefn = pref | (binsel.astype(jnp.uint32) << jnp.uint32(sh))
            kremn = (krem - above).astype(jnp.uint32)
            row_v[...] = jnp.where(lane == 0, prefn,
                                   jnp.where(lane == 1, kremn, jnp.uint32(0)))
            pltpu.sync_copy(row_v, sh_ctl.at[bb, pl.ds(0, _LN)])

        plsc.subcore_barrier()
        pltpu.sync_copy(sh_ctl, ctl_v)
        plsc.subcore_barrier()

    def eqcnt(bb, vec):
        kstar = ctl_v[bb, pl.ds(0, _LN)][0]

        @plsc.parallel_loop(0, rps, carry=jnp.int32(0))
        def cnt(r, c):
            for g in range(w16):
                kv = data_v[bb, r, pl.ds(g * _LN, _LN)]
                c = c + jnp.sum((kv == kstar).astype(jnp.int32))
            return c
        return jnp.where(lane == bb, cnt, vec)

    eqc_loc_v[...] = lax.fori_loop(0, bpc, eqcnt, jnp.zeros((_LN,), jnp.int32))
    pltpu.sync_copy(eqc_loc_v, sh_eqc.at[sid, pl.ds(0, _LN)])
    plsc.subcore_barrier()
    pltpu.sync_copy(sh_eqc, eqc_all_v)

    def pb_body(ws, c):
        return c + jnp.where(ws < sid, eqc_all_v[ws, pl.ds(0, _LN)], 0)

    pb_vec = lax.fori_loop(0, _NS, pb_body, jnp.zeros((_LN,), jnp.int32))

    def kremv(bb, vec):
        return jnp.where(lane == bb, ctl_v[bb, pl.ds(0, _LN)][1].astype(jnp.int32), vec)

    krem_vec = lax.fori_loop(0, bpc, kremv, jnp.zeros((_LN,), jnp.int32))
    nl_v[...] = jnp.clip(krem_vec - pb_vec, 0, eqc_loc_v[...])

    nl_vec = nl_v[...]
    for bb in range(bpc):
        kstar = ctl_v[bb, pl.ds(0, _LN)][0]
        nl = nl_vec[bb]

        @plsc.parallel_loop(0, rps, carry=jnp.int32(0))
        def _mask_r(r, cnt, bb=bb, kstar=kstar, nl=nl):
            for g in range(w16):
                sl = pl.ds(g * _LN, _LN)
                kv = data_v[bb, r, sl]
                eq = kv == kstar
                eqi = eq.astype(jnp.int32)
                excl = plsc.cumsum(eqi) - eqi + cnt
                sel = (kv > kstar) | (eq & (excl < nl))
                mask_v[bb, r, sl] = jnp.where(sel, 1.0, 0.0)
                cnt = cnt + jnp.sum(eqi)
            return cnt

    pltpu.sync_copy(mask_v, mask_hbm.at[pl.ds(b0, bpc), pl.ds(r0, rps), :])


def _sc_select(keys, k):
    b, h, w = keys.shape
    bpc = b // _NC
    rps = h // _NS
    nbin = 256
    mesh = plsc.VectorSubcoreMesh(core_axis_name="c", subcore_axis_name="s")
    return pl.kernel(
        functools.partial(_sc_select_body, bpc=bpc, rps=rps, w=w, k=k),
        mesh=mesh,
        compiler_params=pltpu.CompilerParams(needs_layout_passes=False),
        out_type=jax.ShapeDtypeStruct((b, h, w), jnp.float32),
        scratch_types=[
            pltpu.VMEM((bpc, rps, w), jnp.uint32),
            pltpu.VMEM((bpc, rps, w), jnp.float32),
            pltpu.VMEM((bpc * nbin,), jnp.int32),
            pltpu.VMEM((bpc * nbin,), jnp.int32),
            pltpu.VMEM((_NS, nbin), jnp.int32),
            pltpu.VMEM((nbin,), jnp.int32),
            pltpu.VMEM((bpc, _CTLW), jnp.uint32),
            pltpu.VMEM((_LN,), jnp.int32),
            pltpu.VMEM((_NS, _CTLW), jnp.int32),
            pltpu.VMEM((_LN,), jnp.int32),
            pltpu.VMEM((_LN,), jnp.uint32),
            pltpu.VMEM_SHARED((_NS, bpc * nbin), jnp.int32),
            pltpu.VMEM_SHARED((bpc, _CTLW), jnp.uint32),
            pltpu.VMEM_SHARED((_NS, _CTLW), jnp.int32),
        ],
    )(keys)


def _apply_body(x_ref, m_ref, o_ref):
    o_ref[...] = x_ref[...] * m_ref[...][:, None]


def kernel(x):
    b, c, h, w = x.shape
    hw = h * w
    k = int(hw * MASK_RATIO)
    hchunks = 8
    hblk = h // hchunks

    keys = pl.pallas_call(
        functools.partial(_energy_body, inv_c=1.0 / c),
        grid=(b, hchunks),
        in_specs=[pl.BlockSpec((1, c, hblk, w), lambda i, s: (i, 0, s, 0))],
        out_specs=pl.BlockSpec((1, hblk, w), lambda i, s: (i, s, 0)),
        out_shape=jax.ShapeDtypeStruct((b, h, w), jnp.uint32),
    )(x)

    mask = _sc_select(keys, k)

    out = pl.pallas_call(
        _apply_body,
        grid=(b, hchunks),
        in_specs=[
            pl.BlockSpec((1, c, hblk, w), lambda i, s: (i, 0, s, 0)),
            pl.BlockSpec((1, hblk, w), lambda i, s: (i, s, 0)),
        ],
        out_specs=pl.BlockSpec((1, c, hblk, w), lambda i, s: (i, 0, s, 0)),
        out_shape=jax.ShapeDtypeStruct((b, c, h, w), jnp.float32),
    )(x, mask)

    return out


# speedup vs baseline: 1.0281x; 1.0281x over previous
import functools

import jax
import jax.numpy as jnp
from jax import lax
from jax.experimental import pallas as pl
from jax.experimental.pallas import tpu as pltpu
from jax.experimental.pallas import tpu_sc as plsc

MASK_RATIO = 0.5
_NC = 2
_NS = 16
_LN = 16
_CTLW = 64


def _energy_body(x_ref, e_ref, *, inv_c):
    e = jnp.sum(x_ref[0], axis=0) * inv_c
    bu = lax.bitcast_convert_type(e, jnp.uint32)
    key = jnp.where(bu >= jnp.uint32(0x80000000), ~bu,
                    bu | jnp.uint32(0x80000000))
    e_ref[...] = key[None]


def _sc_select_body(keys_hbm, mask_hbm, data_v, mask_v, hist_v, hist2_v,
                    lead_v, hsum_v, ctl_v, eqc_loc_v, eqc_all_v, nl_v, row_v,
                    sh_hist, sh_ctl, sh_eqc, *, bpc, rps, w, k):
    w16 = w // _LN
    nbin = 256
    sid = lax.axis_index("s")
    core = lax.axis_index("c")
    b0 = core * bpc
    r0 = sid * rps
    lane = lax.iota(jnp.int32, _LN)

    pltpu.sync_copy(keys_hbm.at[pl.ds(b0, bpc), pl.ds(r0, rps), :], data_v)

    for t in range(4):
        sh = 24 - 8 * t
        for j in range((bpc * nbin) // _LN):
            hist_v[pl.ds(j * _LN, _LN)] = jnp.zeros((_LN,), jnp.int32)

        def scan_bb(bb, _, t=t, sh=sh):
            if t > 0:
                pref_hi = ctl_v[bb, pl.ds(0, _LN)][0] >> jnp.uint32(sh + 8)

            def scan_r(r, _):
                for g in range(w16):
                    kv = data_v[bb, r, pl.ds(g * _LN, _LN)]
                    digits = ((kv >> jnp.uint32(sh)) & jnp.uint32(nbin - 1)
                              ).astype(jnp.int32) + bb * nbin
                    if t > 0:
                        match = (kv >> jnp.uint32(sh + 8)) == pref_hi
                        cnts, lastm = plsc.scan_count(digits, match)
                    else:
                        cnts, lastm = plsc.scan_count(digits)
                    plsc.addupdate_scatter(hist_v, [digits], cnts, mask=lastm)
                return 0

            lax.fori_loop(0, rps, scan_r, 0, unroll=2)
            return 0

        lax.fori_loop(0, bpc, scan_bb, 0)

        for j in range((bpc * nbin) // _LN):
            sl = pl.ds(j * _LN, _LN)
            hist2_v[sl] = hist_v[sl]
        pltpu.sync_copy(hist2_v, sh_hist.at[sid])
        plsc.subcore_barrier()

        @pl.when(sid < bpc)
        def _leader(t=t, sh=sh):
            bb = sid
            pltpu.sync_copy(sh_hist.at[:, pl.ds(bb * nbin, nbin)], lead_v)
            for j in range(nbin // _LN):
                hsum_v[pl.ds(j * _LN, _LN)] = jnp.zeros((_LN,), jnp.int32)

            def accum(rr, _):
                for j in range(nbin // _LN):
                    sl = pl.ds(j * _LN, _LN)
                    hsum_v[sl] = hsum_v[sl] + lead_v[rr, sl]
                return 0

            lax.fori_loop(0, _NS, accum, 0)

            if t == 0:
                pref = jnp.uint32(0)
                krem = jnp.int32(k)
            else:
                rowc = ctl_v[bb, pl.ds(0, _LN)]
                pref = rowc[0]
                krem = rowc[1].astype(jnp.int32)

            def suffix(j, carry):
                acc, found, binsel, above = carry
                start = (nbin // _LN - 1 - j) * _LN
                v = hsum_v[pl.ds(start, _LN)]
                rv = lax.rev(v, (0,))
                s = plsc.cumsum(rv) + acc
                mlt = s < krem
                nlt = jnp.sum(mlt.astype(jnp.int32))
                tot = jnp.sum(v)
                hit = ((acc + tot) >= krem) & jnp.logical_not(found)
                bin_new = start + (_LN - 1) - nlt
                above_new = acc + jnp.sum(jnp.where(mlt, rv, 0))
                return (acc + tot, found | hit,
                        jnp.where(hit, bin_new, binsel),
                        jnp.where(hit, above_new, above))

            _, _, binsel, above = lax.fori_loop(
                0, nbin // _LN, suffix,
                (jnp.int32(0), False, jnp.int32(0), jnp.int32(0)))
            prefn = pref | (binsel.astype(jnp.uint32) << jnp.uint32(sh))
            kremn = (krem - above).astype(jnp.uint32)
            row_v[...] = jnp.where(lane == 0, prefn,
                                   jnp.where(lane == 1, kremn, jnp.uint32(0)))
            pltpu.sync_copy(row_v, sh_ctl.at[bb, pl.ds(0, _LN)])

        plsc.subcore_barrier()
        pltpu.sync_copy(sh_ctl, ctl_v)
        plsc.subcore_barrier()

    def eqidx(bb, vec):
        kstar = ctl_v[bb, pl.ds(0, _LN)][0]
        ei = bb * nbin + (kstar & jnp.uint32(0xFF)).astype(jnp.int32)
        return jnp.where(lane == bb, ei, vec)

    eq_idx = lax.fori_loop(0, bpc, eqidx, jnp.zeros((_LN,), jnp.int32))
    eqc_loc_v[...] = jnp.where(lane < bpc,
                               plsc.load_gather(hist_v, [eq_idx]), 0)
    pltpu.sync_copy(eqc_loc_v, sh_eqc.at[sid, pl.ds(0, _LN)])
    plsc.subcore_barrier()
    pltpu.sync_copy(sh_eqc, eqc_all_v)

    def pb_body(ws, c):
        return c + jnp.where(ws < sid, eqc_all_v[ws, pl.ds(0, _LN)], 0)

    pb_vec = lax.fori_loop(0, _NS, pb_body, jnp.zeros((_LN,), jnp.int32))

    def kremv(bb, vec):
        return jnp.where(lane == bb, ctl_v[bb, pl.ds(0, _LN)][1].astype(jnp.int32), vec)

    krem_vec = lax.fori_loop(0, bpc, kremv, jnp.zeros((_LN,), jnp.int32))
    nl_v[...] = jnp.clip(krem_vec - pb_vec, 0, eqc_loc_v[...])

    nl_vec = nl_v[...]
    for bb in range(bpc):
        kstar = ctl_v[bb, pl.ds(0, _LN)][0]
        nl = nl_vec[bb]

        @plsc.parallel_loop(0, rps, carry=jnp.int32(0))
        def _mask_r(r, cnt, bb=bb, kstar=kstar, nl=nl):
            for g in range(w16):
                sl = pl.ds(g * _LN, _LN)
                kv = data_v[bb, r, sl]
                eq = kv == kstar
                eqi = eq.astype(jnp.int32)
                excl = plsc.cumsum(eqi) - eqi + cnt
                sel = (kv > kstar) | (eq & (excl < nl))
                mask_v[bb, r, sl] = jnp.where(sel, 1.0, 0.0)
                cnt = cnt + jnp.sum(eqi)
            return cnt

    pltpu.sync_copy(mask_v, mask_hbm.at[pl.ds(b0, bpc), pl.ds(r0, rps), :])


def _sc_select(keys, k):
    b, h, w = keys.shape
    bpc = b // _NC
    rps = h // _NS
    nbin = 256
    mesh = plsc.VectorSubcoreMesh(core_axis_name="c", subcore_axis_name="s")
    return pl.kernel(
        functools.partial(_sc_select_body, bpc=bpc, rps=rps, w=w, k=k),
        mesh=mesh,
        compiler_params=pltpu.CompilerParams(needs_layout_passes=False),
        out_type=jax.ShapeDtypeStruct((b, h, w), jnp.float32),
        scratch_types=[
            pltpu.VMEM((bpc, rps, w), jnp.uint32),
            pltpu.VMEM((bpc, rps, w), jnp.float32),
            pltpu.VMEM((bpc * nbin,), jnp.int32),
            pltpu.VMEM((bpc * nbin,), jnp.int32),
            pltpu.VMEM((_NS, nbin), jnp.int32),
            pltpu.VMEM((nbin,), jnp.int32),
            pltpu.VMEM((bpc, _CTLW), jnp.uint32),
            pltpu.VMEM((_LN,), jnp.int32),
            pltpu.VMEM((_NS, _CTLW), jnp.int32),
            pltpu.VMEM((_LN,), jnp.int32),
            pltpu.VMEM((_LN,), jnp.uint32),
            pltpu.VMEM_SHARED((_NS, bpc * nbin), jnp.int32),
            pltpu.VMEM_SHARED((bpc, _CTLW), jnp.uint32),
            pltpu.VMEM_SHARED((_NS, _CTLW), jnp.int32),
        ],
    )(keys)


def _apply_body(x_ref, m_ref, o_ref):
    o_ref[...] = x_ref[...] * m_ref[...][:, None]


def kernel(x):
    b, c, h, w = x.shape
    hw = h * w
    k = int(hw * MASK_RATIO)
    hchunks = 8
    hblk = h // hchunks

    keys = pl.pallas_call(
        functools.partial(_energy_body, inv_c=1.0 / c),
        grid=(b, hchunks),
        in_specs=[pl.BlockSpec((1, c, hblk, w), lambda i, s: (i, 0, s, 0))],
        out_specs=pl.BlockSpec((1, hblk, w), lambda i, s: (i, s, 0)),
        out_shape=jax.ShapeDtypeStruct((b, h, w), jnp.uint32),
    )(x)

    mask = _sc_select(keys, k)

    out = pl.pallas_call(
        _apply_body,
        grid=(b, hchunks),
        in_specs=[
            pl.BlockSpec((1, c, hblk, w), lambda i, s: (i, 0, s, 0)),
            pl.BlockSpec((1, hblk, w), lambda i, s: (i, s, 0)),
        ],
        out_specs=pl.BlockSpec((1, c, hblk, w), lambda i, s: (i, 0, s, 0)),
        out_shape=jax.ShapeDtypeStruct((b, c, h, w), jnp.float32),
    )(x, mask)

    return out
